# baseline (speedup 1.0000x reference)
import functools

import jax
import jax.numpy as jnp
from jax import lax
from jax.experimental import pallas as pl
from jax.experimental.pallas import tpu as pltpu, tpu_sc as plsc

TOPK = 16
N = 4096
B = 128
R = 256
NSEG = 4
Q = N // NSEG
NW = 32
ROWS_W = N // NW
CB = 16
NSLAB = B // CB


def _extract_body(m_ref, idx_ref, w_ref):
    colio = jax.lax.broadcasted_iota(jnp.int32, (R, Q), 1)
    vs = [m_ref[:, s * Q : (s + 1) * Q] for s in range(NSEG)]
    ks = [colio + s * Q for s in range(NSEG)]

    def ce(i, j):
        vi, vj = vs[i], vs[j]
        ki, kj = ks[i], ks[j]
        swap = (vi < vj) | ((vi == vj) & (ki > kj))
        vs[i] = jnp.where(swap, vj, vi)
        vs[j] = jnp.where(swap, vi, vj)
        ks[i] = jnp.where(swap, kj, ki)
        ks[j] = jnp.where(swap, ki, kj)

    for i, j in [(0, 2), (1, 3), (0, 1), (2, 3), (1, 2)]:
        ce(i, j)
    v0, v1, v2, v3 = vs
    k0, k1, k2, k3 = ks
    args = []
    for _ in range(TOPK):
        cur = jnp.max(v0, axis=1, keepdims=True)
        cand = jnp.where(v0 == cur, k0, N)
        cstar = jnp.min(cand, axis=1, keepdims=True)
        args.append(cstar)
        sel = cand == cstar
        v0 = jnp.where(sel, v1, v0)
        k0 = jnp.where(sel, k1, k0)
        v1 = jnp.where(sel, v2, v1)
        k1 = jnp.where(sel, k2, k1)
        v2 = jnp.where(sel, v3, v2)
        k2 = jnp.where(sel, k3, k2)
        v3 = jnp.where(sel, -jnp.inf, v3)
    idx_ref[...] = jnp.concatenate(args, axis=1)
    w_ref[...] = m_ref[:, :TOPK]


def _extract(mask):
    return pl.pallas_call(
        _extract_body,
        grid=(N // R,),
        in_specs=[pl.BlockSpec((R, N), lambda g: (g, 0))],
        out_specs=[
            pl.BlockSpec((R, TOPK), lambda g: (g, 0)),
            pl.BlockSpec((R, TOPK), lambda g: (g, 0)),
        ],
        out_shape=[
            jax.ShapeDtypeStruct((N, TOPK), jnp.int32),
            jax.ShapeDtypeStruct((N, TOPK), jnp.float32),
        ],
    )(mask)


@functools.partial(
    pl.kernel,
    mesh=plsc.VectorSubcoreMesh(core_axis_name="c", subcore_axis_name="s"),
    compiler_params=pltpu.CompilerParams(needs_layout_passes=False),
    out_type=jax.ShapeDtypeStruct((NW, B, ROWS_W), jnp.float32),
    scratch_types=[
        pltpu.VMEM((CB * N,), jnp.float32),
        pltpu.VMEM((TOPK, ROWS_W), jnp.int32),
        pltpu.VMEM((TOPK, ROWS_W), jnp.float32),
        pltpu.VMEM((ROWS_W,), jnp.float32),
        pltpu.VMEM((CB, ROWS_W), jnp.float32),
    ],
)
def _sc_contract(x_hbm, idx3_hbm, w3_hbm, bias2_hbm, out3_hbm,
                 slab, idx_v, w_v, bias_v, obuf):
    cid = lax.axis_index("c")
    sid = lax.axis_index("s")
    wid = sid * 2 + cid
    pltpu.sync_copy(idx3_hbm.at[wid], idx_v)
    pltpu.sync_copy(w3_hbm.at[wid], w_v)
    pltpu.sync_copy(bias2_hbm.at[wid], bias_v)

    def slab_body(s, _):
        pltpu.sync_copy(x_hbm.at[pl.ds(s * (CB * N), CB * N)], slab)
        for b in range(CB):
            for g in range(ROWS_W // 16):
                rb = g * 16
                acc = bias_v[pl.ds(rb, 16)]
                for j in range(TOPK):
                    idxj = idx_v[j, pl.ds(rb, 16)]
                    wj = w_v[j, pl.ds(rb, 16)]
                    acc = acc + wj * plsc.load_gather(slab, [idxj + b * N])
                obuf[b, pl.ds(rb, 16)] = acc
        pltpu.sync_copy(obuf, out3_hbm.at[wid, pl.ds(s * CB, CB)])
        return 0

    lax.fori_loop(0, NSLAB, slab_body, 0)


@jax.jit
def kernel(x, mask, bias):
    idx, w = _extract(mask)
    idx3 = idx.T.reshape(TOPK, NW, ROWS_W).transpose(1, 0, 2)
    w3 = w.T.reshape(TOPK, NW, ROWS_W).transpose(1, 0, 2)
    bias2 = bias.reshape(NW, ROWS_W)
    out3 = _sc_contract(x.reshape(-1), idx3, w3, bias2)
    return out3.transpose(1, 0, 2).reshape(B, N)


# speedup vs baseline: 1.2378x; 1.2378x over previous
import functools

import jax
import jax.numpy as jnp
from jax import lax
from jax.experimental import pallas as pl
from jax.experimental.pallas import tpu as pltpu, tpu_sc as plsc

TOPK = 16
N = 4096
B = 128
R = 256
NSEG = 4
Q = N // NSEG
NW = 32
ROWS_W = N // NW
CB = 8
NSLAB = B // CB


def _extract_body(m_ref, idx_ref, w_ref):
    colio = jax.lax.broadcasted_iota(jnp.int32, (R, Q), 1)
    vs = [m_ref[:, s * Q : (s + 1) * Q] for s in range(NSEG)]
    ks = [colio + s * Q for s in range(NSEG)]

    def ce(i, j):
        vi, vj = vs[i], vs[j]
        ki, kj = ks[i], ks[j]
        swap = (vi < vj) | ((vi == vj) & (ki > kj))
        vs[i] = jnp.where(swap, vj, vi)
        vs[j] = jnp.where(swap, vi, vj)
        ks[i] = jnp.where(swap, kj, ki)
        ks[j] = jnp.where(swap, ki, kj)

    for i, j in [(0, 2), (1, 3), (0, 1), (2, 3), (1, 2)]:
        ce(i, j)
    v0, v1, v2, v3 = vs
    k0, k1, k2, k3 = ks
    args = []
    for _ in range(TOPK):
        cur = jnp.max(v0, axis=1, keepdims=True)
        cand = jnp.where(v0 == cur, k0, N)
        cstar = jnp.min(cand, axis=1, keepdims=True)
        args.append(cstar)
        sel = cand == cstar
        v0 = jnp.where(sel, v1, v0)
        k0 = jnp.where(sel, k1, k0)
        v1 = jnp.where(sel, v2, v1)
        k1 = jnp.where(sel, k2, k1)
        v2 = jnp.where(sel, v3, v2)
        k2 = jnp.where(sel, k3, k2)
        v3 = jnp.where(sel, -jnp.inf, v3)
    idx_ref[...] = jnp.concatenate(args, axis=1)
    w_ref[...] = m_ref[:, :TOPK]


def _extract(mask):
    return pl.pallas_call(
        _extract_body,
        grid=(N // R,),
        in_specs=[pl.BlockSpec((R, N), lambda g: (g, 0))],
        out_specs=[
            pl.BlockSpec((R, TOPK), lambda g: (g, 0)),
            pl.BlockSpec((R, TOPK), lambda g: (g, 0)),
        ],
        out_shape=[
            jax.ShapeDtypeStruct((N, TOPK), jnp.int32),
            jax.ShapeDtypeStruct((N, TOPK), jnp.float32),
        ],
    )(mask)


@functools.partial(
    pl.kernel,
    mesh=plsc.VectorSubcoreMesh(core_axis_name="c", subcore_axis_name="s"),
    compiler_params=pltpu.CompilerParams(needs_layout_passes=False),
    out_type=jax.ShapeDtypeStruct((NW, B, ROWS_W), jnp.float32),
    scratch_types=[
        pltpu.VMEM((CB * N,), jnp.float32),
        pltpu.VMEM((CB * N,), jnp.float32),
        pltpu.VMEM((TOPK, ROWS_W), jnp.int32),
        pltpu.VMEM((TOPK, ROWS_W), jnp.float32),
        pltpu.VMEM((ROWS_W,), jnp.float32),
        pltpu.VMEM((CB, ROWS_W), jnp.float32),
        pltpu.VMEM((CB, ROWS_W), jnp.float32),
        pltpu.SemaphoreType.DMA,
        pltpu.SemaphoreType.DMA,
        pltpu.SemaphoreType.DMA,
        pltpu.SemaphoreType.DMA,
    ],
)
def _sc_contract(x_hbm, idx3_hbm, w3_hbm, bias2_hbm, out3_hbm,
                 slabA, slabB, idx_v, w_v, bias_v, obufA, obufB,
                 semA, semB, semOA, semOB):
    cid = lax.axis_index("c")
    sid = lax.axis_index("s")
    wid = sid * 2 + cid
    pltpu.sync_copy(idx3_hbm.at[wid], idx_v)
    pltpu.sync_copy(w3_hbm.at[wid], w_v)
    pltpu.sync_copy(bias2_hbm.at[wid], bias_v)

    def xslice(s):
        return x_hbm.at[pl.ds(s * (CB * N), CB * N)]

    def compute(slab, obuf, g, _):
        rb = g * 16
        idxs = [idx_v[j, pl.ds(rb, 16)] for j in range(TOPK)]
        ws = [w_v[j, pl.ds(rb, 16)] for j in range(TOPK)]
        bv = bias_v[pl.ds(rb, 16)]
        for b in range(CB):
            acc = bv
            for j in range(TOPK):
                acc = acc + ws[j] * plsc.load_gather(slab, [idxs[j] + b * N])
            obuf[b, pl.ds(rb, 16)] = acc
        return 0

    pltpu.async_copy(xslice(0), slabA, semA)

    def pair_body(i, _):
        sA = 2 * i
        pltpu.async_copy(xslice(sA + 1), slabB, semB)
        pltpu.make_async_copy(xslice(sA), slabA, semA).wait()

        @pl.when(i > 0)
        def _():
            pltpu.make_async_copy(obufA, out3_hbm.at[wid, pl.ds(0, CB)],
                                  semOA).wait()

        lax.fori_loop(0, ROWS_W // 16,
                      functools.partial(compute, slabA, obufA), 0)
        pltpu.async_copy(obufA, out3_hbm.at[wid, pl.ds(sA * CB, CB)], semOA)

        @pl.when(i < NSLAB // 2 - 1)
        def _():
            pltpu.async_copy(xslice(sA + 2), slabA, semA)

        pltpu.make_async_copy(xslice(sA + 1), slabB, semB).wait()

        @pl.when(i > 0)
        def _():
            pltpu.make_async_copy(obufB, out3_hbm.at[wid, pl.ds(0, CB)],
                                  semOB).wait()

        lax.fori_loop(0, ROWS_W // 16,
                      functools.partial(compute, slabB, obufB), 0)
        pltpu.async_copy(obufB, out3_hbm.at[wid, pl.ds((sA + 1) * CB, CB)],
                         semOB)
        return 0

    lax.fori_loop(0, NSLAB // 2, pair_body, 0)
    pltpu.make_async_copy(obufA, out3_hbm.at[wid, pl.ds(0, CB)], semOA).wait()
    pltpu.make_async_copy(obufB, out3_hbm.at[wid, pl.ds(0, CB)], semOB).wait()


@jax.jit
def kernel(x, mask, bias):
    idx, w = _extract(mask)
    idx3 = idx.T.reshape(TOPK, NW, ROWS_W).transpose(1, 0, 2)
    w3 = w.T.reshape(TOPK, NW, ROWS_W).transpose(1, 0, 2)
    bias2 = bias.reshape(NW, ROWS_W)
    out3 = _sc_contract(x.reshape(-1), idx3, w3, bias2)
    return out3.transpose(1, 0, 2).reshape(B, N)


# speedup vs baseline: 1.2663x; 1.0230x over previous
import functools

import jax
import jax.numpy as jnp
from jax import lax
from jax.experimental import pallas as pl
from jax.experimental.pallas import tpu as pltpu, tpu_sc as plsc

TOPK = 16
N = 4096
B = 128
R = 256
NSEG = 4
Q = N // NSEG
NW = 32
ROWS_W = N // NW
CB = 8
NSLAB = B // CB


def _extract_body(m_ref, idx_ref, w_ref):
    colio = jax.lax.broadcasted_iota(jnp.int32, (R, Q), 1).astype(jnp.float32)
    vs = [m_ref[:, s * Q : (s + 1) * Q] for s in range(NSEG)]
    ks = [colio + float(s * Q) for s in range(NSEG)]

    def ce(i, j):
        vi, vj = vs[i], vs[j]
        ki, kj = ks[i], ks[j]
        swap = (vi < vj) | ((vi == vj) & (ki > kj))
        vs[i] = jnp.where(swap, vj, vi)
        vs[j] = jnp.where(swap, vi, vj)
        ks[i] = jnp.where(swap, kj, ki)
        ks[j] = jnp.where(swap, ki, kj)

    for i, j in [(0, 2), (1, 3), (0, 1), (2, 3), (1, 2)]:
        ce(i, j)
    v0, v1, v2, v3 = vs
    k0, k1, k2, k3 = ks
    args = []
    for _ in range(TOPK):
        cur = jnp.max(v0, axis=1, keepdims=True)
        cand = jnp.where(v0 == cur, k0, float(N))
        cstar = jnp.min(cand, axis=1, keepdims=True)
        args.append(cstar)
        sel = cand == cstar
        v0 = jnp.where(sel, v1, v0)
        k0 = jnp.where(sel, k1, k0)
        v1 = jnp.where(sel, v2, v1)
        k1 = jnp.where(sel, k2, k1)
        v2 = jnp.where(sel, v3, v2)
        k2 = jnp.where(sel, k3, k2)
        v3 = jnp.where(sel, -jnp.inf, v3)
    idx_ref[...] = jnp.concatenate(args, axis=1).astype(jnp.int32)
    w_ref[...] = m_ref[:, :TOPK]


def _extract(mask):
    return pl.pallas_call(
        _extract_body,
        grid=(N // R,),
        in_specs=[pl.BlockSpec((R, N), lambda g: (g, 0))],
        out_specs=[
            pl.BlockSpec((R, TOPK), lambda g: (g, 0)),
            pl.BlockSpec((R, TOPK), lambda g: (g, 0)),
        ],
        out_shape=[
            jax.ShapeDtypeStruct((N, TOPK), jnp.int32),
            jax.ShapeDtypeStruct((N, TOPK), jnp.float32),
        ],
    )(mask)


@functools.partial(
    pl.kernel,
    mesh=plsc.VectorSubcoreMesh(core_axis_name="c", subcore_axis_name="s"),
    compiler_params=pltpu.CompilerParams(needs_layout_passes=False),
    out_type=jax.ShapeDtypeStruct((NW, B, ROWS_W), jnp.float32),
    scratch_types=[
        pltpu.VMEM((CB * N,), jnp.float32),
        pltpu.VMEM((CB * N,), jnp.float32),
        pltpu.VMEM((TOPK, ROWS_W), jnp.int32),
        pltpu.VMEM((TOPK, ROWS_W), jnp.float32),
        pltpu.VMEM((ROWS_W,), jnp.float32),
        pltpu.VMEM((CB, ROWS_W), jnp.float32),
        pltpu.VMEM((CB, ROWS_W), jnp.float32),
        pltpu.SemaphoreType.DMA,
        pltpu.SemaphoreType.DMA,
        pltpu.SemaphoreType.DMA,
        pltpu.SemaphoreType.DMA,
    ],
)
def _sc_contract(x_hbm, idx3_hbm, w3_hbm, bias2_hbm, out3_hbm,
                 slabA, slabB, idx_v, w_v, bias_v, obufA, obufB,
                 semA, semB, semOA, semOB):
    cid = lax.axis_index("c")
    sid = lax.axis_index("s")
    wid = sid * 2 + cid
    pltpu.sync_copy(idx3_hbm.at[wid], idx_v)
    pltpu.sync_copy(w3_hbm.at[wid], w_v)
    pltpu.sync_copy(bias2_hbm.at[wid], bias_v)

    def xslice(s):
        return x_hbm.at[pl.ds(s * (CB * N), CB * N)]

    def compute(slab, obuf, g, _):
        rb = g * 16
        idxs = [idx_v[j, pl.ds(rb, 16)] for j in range(TOPK)]
        ws = [w_v[j, pl.ds(rb, 16)] for j in range(TOPK)]
        bv = bias_v[pl.ds(rb, 16)]
        for b in range(CB):
            acc = bv
            for j in range(TOPK):
                acc = acc + ws[j] * plsc.load_gather(slab, [idxs[j] + b * N])
            obuf[b, pl.ds(rb, 16)] = acc
        return 0

    pltpu.async_copy(xslice(0), slabA, semA)

    def pair_body(i, _):
        sA = 2 * i
        pltpu.async_copy(xslice(sA + 1), slabB, semB)
        pltpu.make_async_copy(xslice(sA), slabA, semA).wait()

        @pl.when(i > 0)
        def _():
            pltpu.make_async_copy(obufA, out3_hbm.at[wid, pl.ds(0, CB)],
                                  semOA).wait()

        lax.fori_loop(0, ROWS_W // 16,
                      functools.partial(compute, slabA, obufA), 0)
        pltpu.async_copy(obufA, out3_hbm.at[wid, pl.ds(sA * CB, CB)], semOA)

        @pl.when(i < NSLAB // 2 - 1)
        def _():
            pltpu.async_copy(xslice(sA + 2), slabA, semA)

        pltpu.make_async_copy(xslice(sA + 1), slabB, semB).wait()

        @pl.when(i > 0)
        def _():
            pltpu.make_async_copy(obufB, out3_hbm.at[wid, pl.ds(0, CB)],
                                  semOB).wait()

        lax.fori_loop(0, ROWS_W // 16,
                      functools.partial(compute, slabB, obufB), 0)
        pltpu.async_copy(obufB, out3_hbm.at[wid, pl.ds((sA + 1) * CB, CB)],
                         semOB)
        return 0

    lax.fori_loop(0, NSLAB // 2, pair_body, 0)
    pltpu.make_async_copy(obufA, out3_hbm.at[wid, pl.ds(0, CB)], semOA).wait()
    pltpu.make_async_copy(obufB, out3_hbm.at[wid, pl.ds(0, CB)], semOB).wait()


@jax.jit
def kernel(x, mask, bias):
    idx, w = _extract(mask)
    idx3 = idx.T.reshape(TOPK, NW, ROWS_W).transpose(1, 0, 2)
    w3 = w.T.reshape(TOPK, NW, ROWS_W).transpose(1, 0, 2)
    bias2 = bias.reshape(NW, ROWS_W)
    out3 = _sc_contract(x.reshape(-1), idx3, w3, bias2)
    return out3.transpose(1, 0, 2).reshape(B, N)
